# baseline (speedup 1.0000x reference)
import functools

import jax
import jax.numpy as jnp
from jax import lax
from jax.experimental import pallas as pl
from jax.experimental.pallas import tpu as pltpu
from jax.experimental.pallas import tpu_sc as plsc

N = 100000
D = 128
G = 64
SCALE = float(1562.5) ** (-0.5)

NC = 2
NS = 16
NW = NC * NS
L = 16
GROUP = 128
BLOCK = 256
GPB = BLOCK // GROUP
NBLK = N // BLOCK
TAILBLK = NBLK
TAIL_ROWS = N - NBLK * BLOCK
TAIL_REM = TAIL_ROWS - GROUP
BPW = -(-(NBLK + 1) // NW)
IDROWS = -(-N // GROUP) + 1
SEGS_PER_TILE = G // NS
C = 8
GX = G * C
GXPT = GX // NS


@functools.partial(
    pl.kernel,
    out_type=jax.ShapeDtypeStruct((NC, G, D), jnp.float32),
    mesh=plsc.VectorSubcoreMesh(core_axis_name="c", subcore_axis_name="s"),
    compiler_params=pltpu.CompilerParams(use_tc_tiling_on_sc=False),
    scratch_types=[
        pltpu.VMEM((2, BLOCK, D), jnp.float32),
        pltpu.VMEM((2, GPB, GROUP), jnp.int32),
        pltpu.VMEM((2, GPB, GROUP), jnp.int32),
        pltpu.VMEM((TAIL_ROWS, D), jnp.float32),
        pltpu.VMEM((GROUP,), jnp.int32),
        pltpu.VMEM((TAIL_REM,), jnp.int32),
        pltpu.VMEM((GROUP,), jnp.int32),
        pltpu.VMEM((TAIL_REM,), jnp.int32),
        pltpu.VMEM((SEGS_PER_TILE * C, D), jnp.float32),
        pltpu.VMEM((SEGS_PER_TILE, D), jnp.float32),
        pltpu.VMEM_SHARED((GX, D), jnp.float32),
        pltpu.SemaphoreType.DMA,
        pltpu.SemaphoreType.DMA,
        pltpu.SemaphoreType.DMA,
        pltpu.SemaphoreType.DMA,
        pltpu.SemaphoreType.DMA,
        pltpu.SemaphoreType.DMA,
    ],
)
def _sc_segment_sum(nodes_ref, ids_ref, zeros_ref, part_ref,
                    nbuf, ibuf, sidx, tnbuf, tidx_a, tidx_b, tsidx_a,
                    tsidx_b, rbuf, outbuf, acc,
                    nsem0, nsem1, isem0, isem1, tnsem, tisem):
    c = lax.axis_index("c")
    s = lax.axis_index("s")
    w = s * NC + c
    nsems = (nsem0, nsem1)
    isems = (isem0, isem1)
    pat = lax.rem(lax.iota(jnp.int32, L), jnp.int32(C))

    pltpu.sync_copy(zeros_ref, acc.at[pl.ds(s * GXPT, GXPT)])
    plsc.subcore_barrier()

    def node_copy(b, slot):
        return pltpu.make_async_copy(
            nodes_ref.at[pl.ds(b * BLOCK, BLOCK)], nbuf.at[slot], nsems[slot])

    def id_copy(b, slot):
        return pltpu.make_async_copy(
            ids_ref.at[pl.ds(b * GPB, GPB)], ibuf.at[slot], isems[slot])

    def tail_copies():
        r0 = NBLK * BLOCK
        return (
            pltpu.make_async_copy(
                nodes_ref.at[pl.ds(r0, TAIL_ROWS)], tnbuf, tnsem),
            pltpu.make_async_copy(ids_ref.at[NBLK * GPB], tidx_a, tisem),
            pltpu.make_async_copy(
                ids_ref.at[NBLK * GPB + 1, pl.ds(0, TAIL_REM)], tidx_b, tisem),
        )

    def start_load(k):
        b = w * BPW + k
        slot = k % 2

        @pl.when(b < NBLK)
        def _():
            node_copy(b, slot).start()
            id_copy(b, slot).start()

        @pl.when(b == TAILBLK)
        def _():
            for cp in tail_copies():
                cp.start()

    def consume(k):
        b = w * BPW + k
        slot = k % 2

        @pl.when(b < NBLK)
        def _():
            node_copy(b, slot).wait()
            id_copy(b, slot).wait()
            for j in range(GPB):
                for m in range(GROUP // L):
                    ids16 = ibuf[slot, j, pl.ds(m * L, L)]
                    sidx[slot, j, pl.ds(m * L, L)] = ids16 * C + pat
            for j in range(GPB):
                pltpu.sync_copy(
                    nbuf.at[slot, pl.ds(j * GROUP, GROUP)],
                    acc.at[sidx.at[slot, j]], add=True)

        @pl.when(b == TAILBLK)
        def _():
            for cp in tail_copies():
                cp.wait()
            for m in range(GROUP // L):
                ids16 = tidx_a[pl.ds(m * L, L)]
                tsidx_a[pl.ds(m * L, L)] = ids16 * C + pat
            for m in range(TAIL_REM // L):
                ids16 = tidx_b[pl.ds(m * L, L)]
                tsidx_b[pl.ds(m * L, L)] = ids16 * C + pat
            pltpu.sync_copy(
                tnbuf.at[pl.ds(0, GROUP)], acc.at[tsidx_a], add=True)
            pltpu.sync_copy(
                tnbuf.at[pl.ds(GROUP, TAIL_REM)], acc.at[tsidx_b], add=True)

    start_load(0)
    for k in range(BPW):
        if k + 1 < BPW:
            start_load(k + 1)
        consume(k)

    plsc.subcore_barrier()

    seg0 = s * SEGS_PER_TILE
    pltpu.sync_copy(acc.at[pl.ds(seg0 * C, SEGS_PER_TILE * C)], rbuf)
    for i in range(SEGS_PER_TILE):
        for l in range(D // L):
            v = rbuf[i * C, pl.ds(l * L, L)]
            for t in range(1, C):
                v = v + rbuf[i * C + t, pl.ds(l * L, L)]
            outbuf[i, pl.ds(l * L, L)] = v
    pltpu.sync_copy(outbuf, part_ref.at[c, pl.ds(seg0, SEGS_PER_TILE)])


def _combine_body(p_ref, o_ref):
    o_ref[...] = (p_ref[0] + p_ref[1]) * SCALE


def kernel(nodes, segment_ids, num_segments):
    ids = segment_ids.astype(jnp.int32)
    ids = jnp.pad(ids, (0, IDROWS * GROUP - N)).reshape(IDROWS, GROUP)
    zeros = jnp.zeros((GXPT, D), jnp.float32)
    partials = _sc_segment_sum(nodes, ids, zeros)
    return pl.pallas_call(
        _combine_body,
        out_shape=jax.ShapeDtypeStruct((G, D), jnp.float32),
    )(partials)


# speedup vs baseline: 1.0747x; 1.0747x over previous
import functools

import jax
import jax.numpy as jnp
from jax import lax
from jax.experimental import pallas as pl
from jax.experimental.pallas import tpu as pltpu
from jax.experimental.pallas import tpu_sc as plsc

N = 100000
D = 128
G = 64
SCALE = float(1562.5) ** (-0.5)

NC = 2
NS = 16
NW = NC * NS
GROUP = 128
BLOCK = 256
GPB = BLOCK // GROUP
NBLK = N // BLOCK
TAILBLK = NBLK
TAIL_ROWS = N - NBLK * BLOCK
TAIL_REM = TAIL_ROWS - GROUP
BPW = -(-(NBLK + 1) // NW)
NPAIR = (BPW + 1) // 2
IDROWS = -(-N // GROUP) + 1
SEGS_PER_TILE = G // NS


@functools.partial(
    pl.kernel,
    out_type=jax.ShapeDtypeStruct((NC, G, D), jnp.float32),
    mesh=plsc.VectorSubcoreMesh(core_axis_name="c", subcore_axis_name="s"),
    compiler_params=pltpu.CompilerParams(use_tc_tiling_on_sc=False),
    scratch_types=[
        pltpu.VMEM((2, BLOCK, D), jnp.float32),
        pltpu.VMEM((2, GPB, GROUP), jnp.int32),
        pltpu.VMEM((TAIL_ROWS, D), jnp.float32),
        pltpu.VMEM((GROUP,), jnp.int32),
        pltpu.VMEM((TAIL_REM,), jnp.int32),
        pltpu.VMEM((SEGS_PER_TILE, D), jnp.float32),
        pltpu.VMEM_SHARED((G, D), jnp.float32),
        pltpu.SemaphoreType.DMA,
        pltpu.SemaphoreType.DMA,
        pltpu.SemaphoreType.DMA,
        pltpu.SemaphoreType.DMA,
        pltpu.SemaphoreType.DMA,
        pltpu.SemaphoreType.DMA,
    ],
)
def _sc_segment_sum(nodes_ref, ids_ref, zeros_ref, part_ref,
                    nbuf, ibuf, tnbuf, tidx_a, tidx_b, outbuf, acc,
                    nsem0, nsem1, isem0, isem1, tnsem, tisem):
    c = lax.axis_index("c")
    s = lax.axis_index("s")
    w = s * NC + c
    nsems = (nsem0, nsem1)
    isems = (isem0, isem1)

    @pl.when(s == 0)
    def _init():
        pltpu.sync_copy(zeros_ref, acc)

    plsc.subcore_barrier()

    def node_copy(b, slot):
        return pltpu.make_async_copy(
            nodes_ref.at[pl.ds(b * BLOCK, BLOCK)], nbuf.at[slot], nsems[slot])

    def id_copy(b, slot):
        return pltpu.make_async_copy(
            ids_ref.at[pl.ds(b * GPB, GPB)], ibuf.at[slot], isems[slot])

    def tail_copies():
        r0 = NBLK * BLOCK
        return (
            pltpu.make_async_copy(
                nodes_ref.at[pl.ds(r0, TAIL_ROWS)], tnbuf, tnsem),
            pltpu.make_async_copy(ids_ref.at[NBLK * GPB], tidx_a, tisem),
            pltpu.make_async_copy(
                ids_ref.at[NBLK * GPB + 1, pl.ds(0, TAIL_REM)], tidx_b, tisem),
        )

    def start_load(k, slot):
        b = w * BPW + k
        in_range = k < BPW

        @pl.when(in_range & (b < NBLK))
        def _():
            node_copy(b, slot).start()
            id_copy(b, slot).start()

        @pl.when(in_range & (b == TAILBLK))
        def _():
            for cp in tail_copies():
                cp.start()

    def consume(k, slot):
        b = w * BPW + k
        in_range = k < BPW

        @pl.when(in_range & (b < NBLK))
        def _():
            node_copy(b, slot).wait()
            id_copy(b, slot).wait()
            for j in range(GPB):
                pltpu.sync_copy(
                    nbuf.at[slot, pl.ds(j * GROUP, GROUP)],
                    acc.at[ibuf.at[slot, j]], add=True)

        @pl.when(in_range & (b == TAILBLK))
        def _():
            for cp in tail_copies():
                cp.wait()
            pltpu.sync_copy(
                tnbuf.at[pl.ds(0, GROUP)], acc.at[tidx_a], add=True)
            pltpu.sync_copy(
                tnbuf.at[pl.ds(GROUP, TAIL_REM)], acc.at[tidx_b], add=True)

    start_load(0, 0)
    start_load(1, 1)

    def pair(i, carry):
        k = i * 2
        consume(k, 0)
        start_load(k + 2, 0)
        consume(k + 1, 1)
        start_load(k + 3, 1)
        return carry

    lax.fori_loop(0, NPAIR, pair, 0)

    plsc.subcore_barrier()

    seg0 = s * SEGS_PER_TILE
    pltpu.sync_copy(acc.at[pl.ds(seg0, SEGS_PER_TILE)], outbuf)
    pltpu.sync_copy(outbuf, part_ref.at[c, pl.ds(seg0, SEGS_PER_TILE)])


def _combine_body(p_ref, o_ref):
    o_ref[...] = (p_ref[0] + p_ref[1]) * SCALE


def kernel(nodes, segment_ids, num_segments):
    ids = segment_ids.astype(jnp.int32)
    ids = jnp.pad(ids, (0, IDROWS * GROUP - N)).reshape(IDROWS, GROUP)
    zeros = jnp.zeros((G, D), jnp.float32)
    partials = _sc_segment_sum(nodes, ids, zeros)
    return pl.pallas_call(
        _combine_body,
        out_shape=jax.ShapeDtypeStruct((G, D), jnp.float32),
    )(partials)
